# Initial kernel scaffold; baseline (speedup 1.0000x reference)
import jax
import jax.numpy as jnp
from jax.experimental import pallas as pl


def kernel(idx, table):
    raise NotImplementedError("write your pallas kernel here")


# speedup vs baseline: 1.0357x; 1.0357x over previous
import functools

import jax
import jax.numpy as jnp
from jax import lax
from jax.experimental import pallas as pl
from jax.experimental.pallas import tpu as pltpu
from jax.experimental.pallas import tpu_sc as plsc

_VOCAB = 1000
_NC = 2
_NS = 16
_NW = _NC * _NS
_CHUNK = 40


@functools.lru_cache(maxsize=None)
def _make_gather(bt: int, vocab: int):
    per_w = bt // _NW
    assert per_w * _NW == bt and per_w % _CHUNK == 0 and per_w % 8 == 0
    nchunk = per_w // _CHUNK
    mesh = plsc.VectorSubcoreMesh(core_axis_name="c", subcore_axis_name="s")

    @functools.partial(
        pl.kernel,
        mesh=mesh,
        compiler_params=pltpu.CompilerParams(use_tc_tiling_on_sc=False),
        out_type=jax.ShapeDtypeStruct((bt, vocab), jnp.float32),
        scratch_types=[
            pltpu.VMEM((per_w,), jnp.int32),
            pltpu.VMEM((_CHUNK, vocab), jnp.float32),
            pltpu.VMEM((_CHUNK, vocab), jnp.float32),
            pltpu.SemaphoreType.DMA,
            pltpu.SemaphoreType.DMA,
        ],
    )
    def gather(idx_hbm, table_hbm, out_hbm, idx_v, rows0, rows1, sem0, sem1):
        wid = lax.axis_index("s") * _NC + lax.axis_index("c")
        base = pl.multiple_of(wid * per_w, 8)
        pltpu.sync_copy(idx_hbm.at[pl.ds(base, per_w)], idx_v)
        pltpu.async_copy(table_hbm.at[idx_v.at[pl.ds(0, _CHUNK)]], rows0, sem0)

        @pl.loop(0, nchunk, step=2)
        def _body(c0):
            o0 = pl.multiple_of(c0 * _CHUNK, 8)
            o1 = pl.multiple_of(o0 + _CHUNK, 8)
            pltpu.async_copy(table_hbm.at[idx_v.at[pl.ds(o1, _CHUNK)]],
                             rows1, sem1)
            pltpu.make_async_copy(table_hbm.at[idx_v.at[pl.ds(o0, _CHUNK)]],
                                  rows0, sem0).wait()
            pltpu.sync_copy(rows0, out_hbm.at[pl.ds(base + o0, _CHUNK)])

            @pl.when(c0 + 2 < nchunk)
            def _():
                o2 = pl.multiple_of(o0 + 2 * _CHUNK, 8)
                pltpu.async_copy(table_hbm.at[idx_v.at[pl.ds(o2, _CHUNK)]],
                                 rows0, sem0)

            pltpu.make_async_copy(table_hbm.at[idx_v.at[pl.ds(o1, _CHUNK)]],
                                  rows1, sem1).wait()
            pltpu.sync_copy(rows1, out_hbm.at[pl.ds(base + o1, _CHUNK)])

    return gather


def kernel(idx, table):
    b, t = idx.shape
    flat = idx.reshape(-1).astype(jnp.int32)
    out = _make_gather(b * t, table.shape[1])(flat, table)
    return out.reshape(b, t, table.shape[1])
